# baseline (speedup 1.0000x reference)
import functools

import jax
import jax.numpy as jnp
from jax import lax
from jax.experimental import pallas as pl
from jax.experimental.pallas import tpu as pltpu
from jax.experimental.pallas import tpu_sc as plsc

_NG = 4
_NS = 2


@functools.lru_cache(maxsize=None)
def _build(hist, batch, dim):
    info = plsc.get_sparse_core_info()
    nc, ns, nl = info.num_cores, info.num_subcores, info.num_lanes
    nw = nc * ns
    assert batch == nw * 128 and dim % 8 == 0 and hist % _NG == 0
    nblk = batch // 128
    ndg = dim // 8

    mesh = plsc.VectorSubcoreMesh(core_axis_name="c", subcore_axis_name="s")

    @functools.partial(
        pl.kernel,
        out_type=jax.ShapeDtypeStruct((hist, ndg, nblk, 8, 128), jnp.float32),
        mesh=mesh,
        scratch_types=[
            pltpu.VMEM((hist, 128), jnp.int32),
            pltpu.VMEM((_NG, 128, dim), jnp.float32),
            pltpu.VMEM((_NS, ndg, 8, 128), jnp.float32),
            [pltpu.SemaphoreType.DMA] * _NG,
            [pltpu.SemaphoreType.DMA] * _NS,
        ],
        compiler_params=pltpu.CompilerParams(
            use_tc_tiling_on_sc=False, needs_layout_passes=False),
    )
    def gather_kernel(idxt_hbm, table_hbm, out_hbm, idx_v, rows_v, tbuf_v,
                      gsem, ssem):
        w = lax.axis_index("s") * nc + lax.axis_index("c")
        pltpu.sync_copy(idxt_hbm.at[:, pl.ds(w * 128, 128)], idx_v)

        lane = lax.iota(jnp.int32, nl)

        def start_gather(t, g):
            pltpu.async_copy(table_hbm.at[idx_v.at[t]], rows_v.at[g], gsem[g])

        def wait_gather(g):
            pltpu.make_async_copy(
                table_hbm.at[idx_v.at[0]], rows_v.at[g], gsem[g]).wait()

        def start_store(t, s):
            pltpu.async_copy(tbuf_v.at[s], out_hbm.at[t, :, w], ssem[s])

        def wait_store(s):
            pltpu.make_async_copy(
                tbuf_v.at[s], out_hbm.at[0, :, w], ssem[s]).wait()

        def transpose(g, s):
            rows = rows_v.at[g]
            tbuf = tbuf_v.at[s]

            def trans_d(d, carry):
                dg = lax.div(d, 8)
                ds = lax.rem(d, 8)
                cidx = jnp.broadcast_to(d, (nl,))
                for j in range(128 // nl):
                    ridx = lane + (j * nl)
                    v = plsc.load_gather(rows, [ridx, cidx])
                    tbuf[dg, ds, pl.ds(j * nl, nl)] = v
                return carry

            lax.fori_loop(0, dim, trans_d, 0)

        def step(t, r, with_gather=True, with_store_wait=True):
            g, s = r % _NG, r % _NS
            wait_gather(g)
            if with_gather:
                start_gather(t + (_NG - 1), (g + _NG - 1) % _NG)
            if with_store_wait:
                wait_store(s)
            transpose(g, s)
            start_store(t, s)

        for g in range(_NG - 1):
            start_gather(g, g)
        for t in range(_NG):
            step(t, t, with_store_wait=(t >= _NS))

        def body(u, carry):
            t0 = u * _NG
            for r in range(_NG):
                step(t0 + r, r)
            return carry

        n_steady = hist // _NG - 2
        lax.fori_loop(1, 1 + n_steady, body, 0)

        t0 = hist - _NG
        for r in range(_NG):
            t = t0 + r
            step(t, r, with_gather=(t + _NG - 1 < hist))
        for s in range(_NS):
            wait_store(s)

    return gather_kernel


def kernel(indices, table):
    batch, hist = indices.shape
    dim = table.shape[1]
    idx_t = indices.T.astype(jnp.int32)
    tmp = _build(hist, batch, dim)(idx_t, table)
    return tmp.transpose(2, 4, 0, 1, 3).reshape(batch, hist, dim)
